# baseline (speedup 1.0000x reference)
import functools

import jax
import jax.numpy as jnp
from jax import lax
from jax.experimental import pallas as pl
from jax.experimental.pallas import tpu as pltpu
from jax.experimental.pallas import tpu_sc as plsc

N_ACC = 10240
PAD_ROW = 10000
HID = 64
NC, NS = 2, 16
NW = NC * NS
C = 256
RPT = N_ACC // NS


def _ceil_div(a, b):
    return -(-a // b)


@functools.lru_cache(maxsize=None)
def _make_gat_sc(e_pad, n_chunks):
    ept = e_pad // NW
    mesh = plsc.VectorSubcoreMesh(core_axis_name="c", subcore_axis_name="s",
                                  num_cores=NC, num_subcores=NS)

    def body(xl_hbm, xr_hbm, src_hbm, dst_hbm, att_hbm,
             acc_out, den_out,
             src_v, dst_v, xl_rows, xr_rows, msg_v, ex_v, att_v, zden_v,
             acc_sh, den_sh, sem1, sem2):
        c = lax.axis_index("c")
        s = lax.axis_index("s")
        wid = c * NS + s
        zero16 = jnp.zeros((16,), jnp.float32)

        def zrow(r, carry):
            for j in range(HID // 16):
                msg_v[r, pl.ds(j * 16, 16)] = zero16
            return carry
        lax.fori_loop(0, C, zrow, 0)

        def zden(i, carry):
            zden_v[pl.ds(i * 16, 16)] = zero16
            return carry
        lax.fori_loop(0, RPT // 16, zden, 0)

        for t in range(RPT // C):
            pltpu.sync_copy(msg_v, acc_sh.at[pl.ds(s * RPT + t * C, C), :])
        pltpu.sync_copy(zden_v, den_sh.at[pl.ds(s * RPT, RPT)])
        pltpu.sync_copy(att_hbm, att_v)
        plsc.subcore_barrier()

        lane = lax.iota(jnp.int32, 16)
        att_vecs = [att_v[pl.ds(j * 16, 16)] for j in range(HID // 16)]

        def chunk(ci, carry):
            base = wid * ept + ci * C
            pltpu.sync_copy(src_hbm.at[pl.ds(base, C)], src_v)
            pltpu.sync_copy(dst_hbm.at[pl.ds(base, C)], dst_v)
            cp1 = pltpu.async_copy(xl_hbm.at[src_v], xl_rows, sem1)
            cp2 = pltpu.async_copy(xr_hbm.at[dst_v], xr_rows, sem2)
            cp1.wait()
            cp2.wait()

            def g_body(g, gcarry):
                eidx = g * 16 + lane
                acc = zero16
                for k in range(HID):
                    kf = jnp.full((16,), k, jnp.int32)
                    a = plsc.load_gather(xl_rows, [eidx, kf])
                    b = plsc.load_gather(xr_rows, [eidx, kf])
                    sv = a + b
                    sv = jnp.maximum(sv, 0.2 * sv)
                    acc = acc + att_vecs[k // 16][k % 16] * sv
                ex = jnp.exp(acc)
                ex_v[pl.ds(g * 16, 16)] = ex
                for k in range(HID):
                    kf = jnp.full((16,), k, jnp.int32)
                    v = plsc.load_gather(xl_rows, [eidx, kf]) * ex
                    plsc.store_scatter(msg_v, [eidx, kf], v)
                return gcarry
            lax.fori_loop(0, C // 16, g_body, 0)

            pltpu.sync_copy(msg_v, acc_sh.at[dst_v], add=True)
            pltpu.sync_copy(ex_v, den_sh.at[dst_v], add=True)
            return carry
        lax.fori_loop(0, n_chunks, chunk, 0)
        plsc.subcore_barrier()

        pltpu.sync_copy(acc_sh.at[pl.ds(s * RPT, RPT), :],
                        acc_out.at[c, pl.ds(s * RPT, RPT), :])
        pltpu.sync_copy(den_sh.at[pl.ds(s * RPT, RPT)],
                        den_out.at[c, pl.ds(s * RPT, RPT)])

    return pl.kernel(
        body,
        out_type=[jax.ShapeDtypeStruct((NC, N_ACC, HID), jnp.float32),
                  jax.ShapeDtypeStruct((NC, N_ACC), jnp.float32)],
        mesh=mesh,
        compiler_params=pltpu.CompilerParams(needs_layout_passes=False,
                                             use_tc_tiling_on_sc=False),
        scratch_types=[
            pltpu.VMEM((C,), jnp.int32),
            pltpu.VMEM((C,), jnp.int32),
            pltpu.VMEM((C, HID), jnp.float32),
            pltpu.VMEM((C, HID), jnp.float32),
            pltpu.VMEM((C, HID), jnp.float32),
            pltpu.VMEM((C,), jnp.float32),
            pltpu.VMEM((HID,), jnp.float32),
            pltpu.VMEM((RPT,), jnp.float32),
            pltpu.VMEM_SHARED((N_ACC, HID), jnp.float32),
            pltpu.VMEM_SHARED((N_ACC,), jnp.float32),
            pltpu.SemaphoreType.DMA,
            pltpu.SemaphoreType.DMA,
        ],
    )


def _gat_edges_sc(xl, xr, src, dst, att):
    e = src.shape[0]
    n_chunks = _ceil_div(e, NW * C)
    e_pad = n_chunks * NW * C
    pad = e_pad - e
    if pad:
        src = jnp.concatenate([src, jnp.zeros((pad,), jnp.int32)])
        dst = jnp.concatenate([dst, jnp.full((pad,), PAD_ROW, jnp.int32)])
    return _make_gat_sc(e_pad, n_chunks)(xl, xr, src, dst, att)


_BLK = 1000


def _dot(a, b):
    return jnp.dot(a, b, preferred_element_type=jnp.float32)


def _full(shape):
    nd = len(shape)
    return pl.BlockSpec(shape, lambda i: (0,) * nd)


def _proj_tc(x, wl, bl, wr, br):
    n, k = x.shape
    h = wl.shape[1]

    def body(x_ref, wl_ref, bl_ref, wr_ref, br_ref, xl_ref, xr_ref):
        xb = x_ref[...]
        xl_ref[...] = _dot(xb, wl_ref[...]) + bl_ref[...]
        xr_ref[...] = _dot(xb, wr_ref[...]) + br_ref[...]

    return pl.pallas_call(
        body,
        grid=(n // _BLK,),
        in_specs=[pl.BlockSpec((_BLK, k), lambda i: (i, 0)),
                  _full((k, h)), _full((1, h)), _full((k, h)), _full((1, h))],
        out_specs=[pl.BlockSpec((_BLK, h), lambda i: (i, 0))] * 2,
        out_shape=[jax.ShapeDtypeStruct((n, h), jnp.float32)] * 2,
    )(x, wl, bl, wr, br)


def _combine(acc_ref, den_ref, bias_ref):
    a = acc_ref[0] + acc_ref[1]
    d = den_ref[0] + den_ref[1]
    return a / (d + 1e-16) + bias_ref[...]


def _combine_proj_tc(acc, den3, bias, wl, bl, wr, br, n):
    h_dim = HID

    def body(acc_ref, den_ref, bias_ref, wl_ref, bl_ref, wr_ref, br_ref,
             xl_ref, xr_ref):
        hb = _combine(acc_ref, den_ref, bias_ref)
        xl_ref[...] = _dot(hb, wl_ref[...]) + bl_ref[...]
        xr_ref[...] = _dot(hb, wr_ref[...]) + br_ref[...]

    return pl.pallas_call(
        body,
        grid=(n // _BLK,),
        in_specs=[pl.BlockSpec((NC, _BLK, HID), lambda i: (0, i, 0)),
                  pl.BlockSpec((NC, _BLK, 1), lambda i: (0, i, 0)),
                  _full((1, HID)),
                  _full((HID, h_dim)), _full((1, h_dim)),
                  _full((HID, h_dim)), _full((1, h_dim))],
        out_specs=[pl.BlockSpec((_BLK, h_dim), lambda i: (i, 0))] * 2,
        out_shape=[jax.ShapeDtypeStruct((n, h_dim), jnp.float32)] * 2,
    )(acc, den3, bias, wl, bl, wr, br)


def _combine_proj2_tc(acc, den3, bias, wl, bl, xf, wr, br, n):
    kf = xf.shape[1]

    def body(acc_ref, den_ref, bias_ref, wl_ref, bl_ref, xf_ref, wr_ref,
             br_ref, xl_ref, xr_ref):
        hb = _combine(acc_ref, den_ref, bias_ref)
        xl_ref[...] = _dot(hb, wl_ref[...]) + bl_ref[...]
        xr_ref[...] = _dot(xf_ref[...], wr_ref[...]) + br_ref[...]

    return pl.pallas_call(
        body,
        grid=(n // _BLK,),
        in_specs=[pl.BlockSpec((NC, _BLK, HID), lambda i: (0, i, 0)),
                  pl.BlockSpec((NC, _BLK, 1), lambda i: (0, i, 0)),
                  _full((1, HID)),
                  _full((HID, HID)), _full((1, HID)),
                  pl.BlockSpec((_BLK, kf), lambda i: (i, 0)),
                  _full((kf, HID)), _full((1, HID))],
        out_specs=[pl.BlockSpec((_BLK, HID), lambda i: (i, 0))] * 2,
        out_shape=[jax.ShapeDtypeStruct((n, HID), jnp.float32)] * 2,
    )(acc, den3, bias, wl, bl, xf, wr, br)


def _final_tc(acc, den3, bias, w1, b1, w2, b2, w3, b3,
              xf, wlin, blin, ws, bs, wc, bc, we, be, n):
    def body(acc_ref, den_ref, bias_ref, w1_ref, b1_ref, w2_ref, b2_ref,
             w3_ref, b3_ref, xf_ref, wlin_ref, blin_ref, ws_ref, bs_ref,
             wc_ref, bc_ref, we_ref, be_ref, out_ref):
        foot = _combine(acc_ref, den_ref, bias_ref)
        m = jnp.maximum(_dot(foot, w1_ref[...]) + b1_ref[...], 0.0)
        m = jnp.maximum(_dot(m, w2_ref[...]) + b2_ref[...], 0.0)
        m3 = _dot(m, w3_ref[...]) + b3_ref[...]
        xb = xf_ref[...]
        lin = _dot(xb, wlin_ref[...]) + blin_ref[...]
        a = jnp.maximum(_dot(xb, ws_ref[...]) + bs_ref[...], 0.0)
        a = jnp.maximum(_dot(a, wc_ref[...]) + bc_ref[...], 0.0)
        a = jnp.maximum(_dot(a, wc_ref[...]) + bc_ref[...], 0.0)
        a = _dot(a, we_ref[...]) + be_ref[...]
        out_ref[...] = lin + a + m3

    kf = xf.shape[1]
    return pl.pallas_call(
        body,
        grid=(n // _BLK,),
        in_specs=[pl.BlockSpec((NC, _BLK, HID), lambda i: (0, i, 0)),
                  pl.BlockSpec((NC, _BLK, 1), lambda i: (0, i, 0)),
                  _full((1, HID)),
                  _full((HID, HID)), _full((1, HID)),
                  _full((HID, HID)), _full((1, HID)),
                  _full((HID, 1)), _full((1, 1)),
                  pl.BlockSpec((_BLK, kf), lambda i: (i, 0)),
                  _full((kf, 1)), _full((1, 1)),
                  _full((kf, HID)), _full((1, HID)),
                  _full((HID, HID)), _full((1, HID)),
                  _full((HID, 1)), _full((1, 1))],
        out_specs=pl.BlockSpec((_BLK, 1), lambda i: (i, 0)),
        out_shape=jax.ShapeDtypeStruct((n, 1), jnp.float32),
    )(acc, den3, bias, w1, b1, w2, b2, w3, b3,
      xf, wlin, blin, ws, bs, wc, bc, we, be)


def kernel(x_pano, x_footprint, edge_index_pano, edge_index_rev, params):
    p = params
    n_pano = x_pano.shape[0]
    n_foot = x_footprint.shape[0]
    src0, dst0 = edge_index_pano[0], edge_index_pano[1]
    srcr, dstr = edge_index_rev[0], edge_index_rev[1]

    def row(b):
        return b.reshape(1, -1)

    xl0, xr0 = _proj_tc(x_pano, p['conv0_Wl'], row(p['conv0_bl']),
                        p['conv0_Wr'], row(p['conv0_br']))
    acc0, den0 = _gat_edges_sc(xl0, xr0, src0, dst0,
                               p['conv0_att'].reshape(HID))
    xl1, xr1 = _combine_proj_tc(acc0, den0[..., None], row(p['conv0_bias']),
                                p['conv1_Wl'], row(p['conv1_bl']),
                                p['conv1_Wr'], row(p['conv1_br']), n_pano)
    acc1, den1 = _gat_edges_sc(xl1, xr1, src0, dst0,
                               p['conv1_att'].reshape(HID))
    xlt, xrt = _combine_proj2_tc(acc1, den1[..., None], row(p['conv1_bias']),
                                 p['convt_Wl'], row(p['convt_bl']),
                                 x_footprint, p['convt_Wr'],
                                 row(p['convt_br']), n_pano)
    acct, dent = _gat_edges_sc(xlt, xrt, srcr, dstr,
                               p['convt_att'].reshape(HID))
    return _final_tc(acct, dent[..., None], row(p['convt_bias']),
                     p['mlp_W1'], row(p['mlp_b1']),
                     p['mlp_W2'], row(p['mlp_b2']),
                     p['mlp_W3'], row(p['mlp_b3']),
                     x_footprint,
                     p['null_W_lin'], row(p['null_b_lin']),
                     p['null_W_s'], row(p['null_b_s']),
                     p['null_W_c'], row(p['null_b_c']),
                     p['null_W_e'], row(p['null_b_e']), n_foot)


# speedup vs baseline: 1.1913x; 1.1913x over previous
import functools

import jax
import jax.numpy as jnp
from jax import lax
from jax.experimental import pallas as pl
from jax.experimental.pallas import tpu as pltpu
from jax.experimental.pallas import tpu_sc as plsc

N_ACC = 10240
PAD_ROW = 10000
HID = 64
NC, NS = 2, 16
NW = NC * NS
C = 128
RPT = N_ACC // NS


def _ceil_div(a, b):
    return -(-a // b)


@functools.lru_cache(maxsize=None)
def _make_gat_sc(e_pad, n_chunks):
    ept = e_pad // NW
    mesh = plsc.VectorSubcoreMesh(core_axis_name="c", subcore_axis_name="s",
                                  num_cores=NC, num_subcores=NS)

    n = n_chunks
    assert n % 4 == 0 and n >= 8

    def body(xl_hbm, xr_hbm, src_hbm, dst_hbm, att_hbm,
             acc_out, den_out,
             src_v0, src_v1, src_v2, src_v3,
             dst_v0, dst_v1, dst_v2, dst_v3,
             xl_r0, xl_r1, xr_r0, xr_r1, msg_v0, msg_v1, ex_v0, ex_v1,
             att_v, zden_v, acc_sh, den_sh,
             si0, si1, si2, si3, sl0, sl1, sr0, sr1, sa0, sa1, sd0, sd1):
        src_vs = [src_v0, src_v1, src_v2, src_v3]
        dst_vs = [dst_v0, dst_v1, dst_v2, dst_v3]
        xl_rows = [xl_r0, xl_r1]
        xr_rows = [xr_r0, xr_r1]
        msg_vs = [msg_v0, msg_v1]
        ex_vs = [ex_v0, ex_v1]
        sem_idx = [si0, si1, si2, si3]
        sem_gl = [sl0, sl1]
        sem_gr = [sr0, sr1]
        sem_sa = [sa0, sa1]
        sem_sd = [sd0, sd1]

        c = lax.axis_index("c")
        s = lax.axis_index("s")
        wid = c * NS + s
        zero16 = jnp.zeros((16,), jnp.float32)

        def zrow(r, carry):
            for j in range(HID // 16):
                msg_v0[r, pl.ds(j * 16, 16)] = zero16
            return carry
        lax.fori_loop(0, C, zrow, 0)

        def zden(i, carry):
            zden_v[pl.ds(i * 16, 16)] = zero16
            return carry
        lax.fori_loop(0, RPT // 16, zden, 0)

        for t in range(RPT // C):
            pltpu.sync_copy(msg_v0, acc_sh.at[pl.ds(s * RPT + t * C, C), :])
        pltpu.sync_copy(zden_v, den_sh.at[pl.ds(s * RPT, RPT)])
        pltpu.sync_copy(att_hbm, att_v)
        plsc.subcore_barrier()

        lane = lax.iota(jnp.int32, 16)
        att_vecs = [att_v[pl.ds(j * 16, 16)] for j in range(HID // 16)]

        def idx_issue(ci, j):
            base = wid * ept + ci * C
            pltpu.async_copy(src_hbm.at[pl.ds(base, C)], src_vs[j], sem_idx[j])
            pltpu.async_copy(dst_hbm.at[pl.ds(base, C)], dst_vs[j], sem_idx[j])

        def idx_wait(ci, j):
            base = wid * ept + ci * C
            pltpu.make_async_copy(src_hbm.at[pl.ds(base, C)], src_vs[j],
                                  sem_idx[j]).wait()
            pltpu.make_async_copy(dst_hbm.at[pl.ds(base, C)], dst_vs[j],
                                  sem_idx[j]).wait()

        def gather_issue(j, b):
            pltpu.async_copy(xl_hbm.at[src_vs[j]], xl_rows[b], sem_gl[b])
            pltpu.async_copy(xr_hbm.at[dst_vs[j]], xr_rows[b], sem_gr[b])

        def gather_wait(j, b):
            pltpu.make_async_copy(xl_hbm.at[src_vs[j]], xl_rows[b],
                                  sem_gl[b]).wait()
            pltpu.make_async_copy(xr_hbm.at[dst_vs[j]], xr_rows[b],
                                  sem_gr[b]).wait()

        def scatter_issue(j, b):
            pltpu.async_copy(msg_vs[b], acc_sh.at[dst_vs[j]], sem_sa[b],
                             add=True)
            pltpu.async_copy(ex_vs[b], den_sh.at[dst_vs[j]], sem_sd[b],
                             add=True)

        def scatter_wait(j, b):
            pltpu.make_async_copy(msg_vs[b], acc_sh.at[dst_vs[j]],
                                  sem_sa[b]).wait()
            pltpu.make_async_copy(ex_vs[b], den_sh.at[dst_vs[j]],
                                  sem_sd[b]).wait()

        def compute(b):
            xl_r, xr_r, msg_r, ex_r = (xl_rows[b], xr_rows[b],
                                       msg_vs[b], ex_vs[b])

            def g_body(g, gcarry):
                eidx = g * 16 + lane
                acc = zero16
                for k in range(HID):
                    kf = jnp.full((16,), k, jnp.int32)
                    a = plsc.load_gather(xl_r, [eidx, kf])
                    bb = plsc.load_gather(xr_r, [eidx, kf])
                    sv = a + bb
                    sv = jnp.maximum(sv, 0.2 * sv)
                    acc = acc + att_vecs[k // 16][k % 16] * sv
                ex = jnp.exp(acc)
                ex_r[pl.ds(g * 16, 16)] = ex
                for k in range(HID):
                    kf = jnp.full((16,), k, jnp.int32)
                    v = plsc.load_gather(xl_r, [eidx, kf]) * ex
                    plsc.store_scatter(msg_r, [eidx, kf], v)
                return gcarry
            lax.fori_loop(0, C // 16, g_body, 0)

        def slot(i, p):
            b = p % 2
            bn = (p + 1) % 2
            jn = (p + 1) % 4
            jnn = (p + 2) % 4

            @pl.when(i >= 2)
            def _():
                scatter_wait(jnn, b)

            @pl.when(i + 2 < n)
            def _():
                idx_issue(i + 2, jnn)

            @pl.when(i + 1 < n)
            def _():
                idx_wait(i + 1, jn)
                gather_issue(jn, bn)

            gather_wait(p, b)
            compute(b)
            scatter_issue(p, b)

        idx_issue(0, 0)
        idx_issue(1, 1)
        idx_wait(0, 0)
        gather_issue(0, 0)

        def quad(q, carry):
            i0 = q * 4
            for p in range(4):
                slot(i0 + p, p)
            return carry
        lax.fori_loop(0, n // 4, quad, 0)

        scatter_wait(2, 0)
        scatter_wait(3, 1)
        plsc.subcore_barrier()

        pltpu.sync_copy(acc_sh.at[pl.ds(s * RPT, RPT), :],
                        acc_out.at[c, pl.ds(s * RPT, RPT), :])
        pltpu.sync_copy(den_sh.at[pl.ds(s * RPT, RPT)],
                        den_out.at[c, pl.ds(s * RPT, RPT)])

    return pl.kernel(
        body,
        out_type=[jax.ShapeDtypeStruct((NC, N_ACC, HID), jnp.float32),
                  jax.ShapeDtypeStruct((NC, N_ACC), jnp.float32)],
        mesh=mesh,
        compiler_params=pltpu.CompilerParams(needs_layout_passes=False,
                                             use_tc_tiling_on_sc=False),
        scratch_types=(
            [pltpu.VMEM((C,), jnp.int32)] * 8 +
            [pltpu.VMEM((C, HID), jnp.float32)] * 6 +
            [pltpu.VMEM((C,), jnp.float32)] * 2 +
            [pltpu.VMEM((HID,), jnp.float32),
             pltpu.VMEM((RPT,), jnp.float32),
             pltpu.VMEM_SHARED((N_ACC, HID), jnp.float32),
             pltpu.VMEM_SHARED((N_ACC,), jnp.float32)] +
            [pltpu.SemaphoreType.DMA] * 12
        ),
    )


def _gat_edges_sc(xl, xr, src, dst, att):
    e = src.shape[0]
    n_chunks = _ceil_div(_ceil_div(e, NW * C), 4) * 4
    e_pad = n_chunks * NW * C
    pad = e_pad - e
    if pad:
        src = jnp.concatenate([src, jnp.zeros((pad,), jnp.int32)])
        dst = jnp.concatenate([dst, jnp.full((pad,), PAD_ROW, jnp.int32)])
    return _make_gat_sc(e_pad, n_chunks)(xl, xr, src, dst, att)


_BLK = 1000


def _dot(a, b):
    return jnp.dot(a, b, preferred_element_type=jnp.float32)


def _full(shape):
    nd = len(shape)
    return pl.BlockSpec(shape, lambda i: (0,) * nd)


def _proj_tc(x, wl, bl, wr, br):
    n, k = x.shape
    h = wl.shape[1]

    def body(x_ref, wl_ref, bl_ref, wr_ref, br_ref, xl_ref, xr_ref):
        xb = x_ref[...]
        xl_ref[...] = _dot(xb, wl_ref[...]) + bl_ref[...]
        xr_ref[...] = _dot(xb, wr_ref[...]) + br_ref[...]

    return pl.pallas_call(
        body,
        grid=(n // _BLK,),
        in_specs=[pl.BlockSpec((_BLK, k), lambda i: (i, 0)),
                  _full((k, h)), _full((1, h)), _full((k, h)), _full((1, h))],
        out_specs=[pl.BlockSpec((_BLK, h), lambda i: (i, 0))] * 2,
        out_shape=[jax.ShapeDtypeStruct((n, h), jnp.float32)] * 2,
    )(x, wl, bl, wr, br)


def _combine(acc_ref, den_ref, bias_ref):
    a = acc_ref[0] + acc_ref[1]
    d = den_ref[0] + den_ref[1]
    return a / (d + 1e-16) + bias_ref[...]


def _combine_proj_tc(acc, den3, bias, wl, bl, wr, br, n):
    h_dim = HID

    def body(acc_ref, den_ref, bias_ref, wl_ref, bl_ref, wr_ref, br_ref,
             xl_ref, xr_ref):
        hb = _combine(acc_ref, den_ref, bias_ref)
        xl_ref[...] = _dot(hb, wl_ref[...]) + bl_ref[...]
        xr_ref[...] = _dot(hb, wr_ref[...]) + br_ref[...]

    return pl.pallas_call(
        body,
        grid=(n // _BLK,),
        in_specs=[pl.BlockSpec((NC, _BLK, HID), lambda i: (0, i, 0)),
                  pl.BlockSpec((NC, _BLK, 1), lambda i: (0, i, 0)),
                  _full((1, HID)),
                  _full((HID, h_dim)), _full((1, h_dim)),
                  _full((HID, h_dim)), _full((1, h_dim))],
        out_specs=[pl.BlockSpec((_BLK, h_dim), lambda i: (i, 0))] * 2,
        out_shape=[jax.ShapeDtypeStruct((n, h_dim), jnp.float32)] * 2,
    )(acc, den3, bias, wl, bl, wr, br)


def _combine_proj2_tc(acc, den3, bias, wl, bl, xf, wr, br, n):
    kf = xf.shape[1]

    def body(acc_ref, den_ref, bias_ref, wl_ref, bl_ref, xf_ref, wr_ref,
             br_ref, xl_ref, xr_ref):
        hb = _combine(acc_ref, den_ref, bias_ref)
        xl_ref[...] = _dot(hb, wl_ref[...]) + bl_ref[...]
        xr_ref[...] = _dot(xf_ref[...], wr_ref[...]) + br_ref[...]

    return pl.pallas_call(
        body,
        grid=(n // _BLK,),
        in_specs=[pl.BlockSpec((NC, _BLK, HID), lambda i: (0, i, 0)),
                  pl.BlockSpec((NC, _BLK, 1), lambda i: (0, i, 0)),
                  _full((1, HID)),
                  _full((HID, HID)), _full((1, HID)),
                  pl.BlockSpec((_BLK, kf), lambda i: (i, 0)),
                  _full((kf, HID)), _full((1, HID))],
        out_specs=[pl.BlockSpec((_BLK, HID), lambda i: (i, 0))] * 2,
        out_shape=[jax.ShapeDtypeStruct((n, HID), jnp.float32)] * 2,
    )(acc, den3, bias, wl, bl, xf, wr, br)


def _final_tc(acc, den3, bias, w1, b1, w2, b2, w3, b3,
              xf, wlin, blin, ws, bs, wc, bc, we, be, n):
    def body(acc_ref, den_ref, bias_ref, w1_ref, b1_ref, w2_ref, b2_ref,
             w3_ref, b3_ref, xf_ref, wlin_ref, blin_ref, ws_ref, bs_ref,
             wc_ref, bc_ref, we_ref, be_ref, out_ref):
        foot = _combine(acc_ref, den_ref, bias_ref)
        m = jnp.maximum(_dot(foot, w1_ref[...]) + b1_ref[...], 0.0)
        m = jnp.maximum(_dot(m, w2_ref[...]) + b2_ref[...], 0.0)
        m3 = _dot(m, w3_ref[...]) + b3_ref[...]
        xb = xf_ref[...]
        lin = _dot(xb, wlin_ref[...]) + blin_ref[...]
        a = jnp.maximum(_dot(xb, ws_ref[...]) + bs_ref[...], 0.0)
        a = jnp.maximum(_dot(a, wc_ref[...]) + bc_ref[...], 0.0)
        a = jnp.maximum(_dot(a, wc_ref[...]) + bc_ref[...], 0.0)
        a = _dot(a, we_ref[...]) + be_ref[...]
        out_ref[...] = lin + a + m3

    kf = xf.shape[1]
    return pl.pallas_call(
        body,
        grid=(n // _BLK,),
        in_specs=[pl.BlockSpec((NC, _BLK, HID), lambda i: (0, i, 0)),
                  pl.BlockSpec((NC, _BLK, 1), lambda i: (0, i, 0)),
                  _full((1, HID)),
                  _full((HID, HID)), _full((1, HID)),
                  _full((HID, HID)), _full((1, HID)),
                  _full((HID, 1)), _full((1, 1)),
                  pl.BlockSpec((_BLK, kf), lambda i: (i, 0)),
                  _full((kf, 1)), _full((1, 1)),
                  _full((kf, HID)), _full((1, HID)),
                  _full((HID, HID)), _full((1, HID)),
                  _full((HID, 1)), _full((1, 1))],
        out_specs=pl.BlockSpec((_BLK, 1), lambda i: (i, 0)),
        out_shape=jax.ShapeDtypeStruct((n, 1), jnp.float32),
    )(acc, den3, bias, w1, b1, w2, b2, w3, b3,
      xf, wlin, blin, ws, bs, wc, bc, we, be)


def kernel(x_pano, x_footprint, edge_index_pano, edge_index_rev, params):
    p = params
    n_pano = x_pano.shape[0]
    n_foot = x_footprint.shape[0]
    src0, dst0 = edge_index_pano[0], edge_index_pano[1]
    srcr, dstr = edge_index_rev[0], edge_index_rev[1]

    def row(b):
        return b.reshape(1, -1)

    xl0, xr0 = _proj_tc(x_pano, p['conv0_Wl'], row(p['conv0_bl']),
                        p['conv0_Wr'], row(p['conv0_br']))
    acc0, den0 = _gat_edges_sc(xl0, xr0, src0, dst0,
                               p['conv0_att'].reshape(HID))
    xl1, xr1 = _combine_proj_tc(acc0, den0[..., None], row(p['conv0_bias']),
                                p['conv1_Wl'], row(p['conv1_bl']),
                                p['conv1_Wr'], row(p['conv1_br']), n_pano)
    acc1, den1 = _gat_edges_sc(xl1, xr1, src0, dst0,
                               p['conv1_att'].reshape(HID))
    xlt, xrt = _combine_proj2_tc(acc1, den1[..., None], row(p['conv1_bias']),
                                 p['convt_Wl'], row(p['convt_bl']),
                                 x_footprint, p['convt_Wr'],
                                 row(p['convt_br']), n_pano)
    acct, dent = _gat_edges_sc(xlt, xrt, srcr, dstr,
                               p['convt_att'].reshape(HID))
    return _final_tc(acct, dent[..., None], row(p['convt_bias']),
                     p['mlp_W1'], row(p['mlp_b1']),
                     p['mlp_W2'], row(p['mlp_b2']),
                     p['mlp_W3'], row(p['mlp_b3']),
                     x_footprint,
                     p['null_W_lin'], row(p['null_b_lin']),
                     p['null_W_s'], row(p['null_b_s']),
                     p['null_W_c'], row(p['null_b_c']),
                     p['null_W_e'], row(p['null_b_e']), n_foot)


# speedup vs baseline: 4.6291x; 3.8857x over previous
import functools

import jax
import jax.numpy as jnp
from jax import lax
from jax.experimental import pallas as pl
from jax.experimental.pallas import tpu as pltpu
from jax.experimental.pallas import tpu_sc as plsc

N_ACC = 10240
PAD_ROW = 10000
HID = 64
NC, NS = 2, 16
NW = NC * NS
C = 128
RPT = N_ACC // NS


def _ceil_div(a, b):
    return -(-a // b)


@functools.lru_cache(maxsize=None)
def _make_gat_sc(e_pad, n_chunks):
    ept = e_pad // NW
    mesh = plsc.VectorSubcoreMesh(core_axis_name="c", subcore_axis_name="s",
                                  num_cores=NC, num_subcores=NS)

    n = n_chunks
    assert n % 4 == 0 and n >= 8

    def body(xl_hbm, xr_hbm, src_hbm, dst_hbm, att_hbm,
             acc_out, den_out,
             src_v0, src_v1, src_v2, src_v3,
             dst_v0, dst_v1, dst_v2, dst_v3,
             xl_r0, xl_r1, xr_r0, xr_r1, msg_v0, msg_v1, ex_v0, ex_v1,
             att_v, zden_v, acc_sh, den_sh,
             si0, si1, si2, si3, sl0, sl1, sr0, sr1, sa0, sa1, sd0, sd1):
        src_vs = [src_v0, src_v1, src_v2, src_v3]
        dst_vs = [dst_v0, dst_v1, dst_v2, dst_v3]
        xl_rows = [xl_r0, xl_r1]
        xr_rows = [xr_r0, xr_r1]
        msg_vs = [msg_v0, msg_v1]
        ex_vs = [ex_v0, ex_v1]
        sem_idx = [si0, si1, si2, si3]
        sem_gl = [sl0, sl1]
        sem_gr = [sr0, sr1]
        sem_sa = [sa0, sa1]
        sem_sd = [sd0, sd1]

        c = lax.axis_index("c")
        s = lax.axis_index("s")
        wid = c * NS + s
        zero16 = jnp.zeros((16,), jnp.float32)

        def zrow(r, carry):
            for off in range(0, HID, 16):
                msg_v0[r, pl.ds(off, 16)] = zero16
            return carry
        lax.fori_loop(0, C, zrow, 0)

        def zden(i, carry):
            zden_v[pl.ds(i * 16, 16)] = zero16
            return carry
        lax.fori_loop(0, RPT // 16, zden, 0)

        for t in range(RPT // C):
            pltpu.sync_copy(msg_v0, acc_sh.at[pl.ds(s * RPT + t * C, C), :])
        pltpu.sync_copy(zden_v, den_sh.at[pl.ds(s * RPT, RPT)])
        pltpu.sync_copy(att_hbm, att_v.at[pl.ds(0, HID)])
        pltpu.sync_copy(att_hbm.at[pl.ds(0, 16)], att_v.at[pl.ds(HID, 16)])
        plsc.subcore_barrier()

        lane = lax.iota(jnp.int32, 16)

        def idx_issue(ci, j):
            base = wid * ept + ci * C
            pltpu.async_copy(src_hbm.at[pl.ds(base, C)], src_vs[j], sem_idx[j])
            pltpu.async_copy(dst_hbm.at[pl.ds(base, C)], dst_vs[j], sem_idx[j])

        def idx_wait(ci, j):
            base = wid * ept + ci * C
            pltpu.make_async_copy(src_hbm.at[pl.ds(base, C)], src_vs[j],
                                  sem_idx[j]).wait()
            pltpu.make_async_copy(dst_hbm.at[pl.ds(base, C)], dst_vs[j],
                                  sem_idx[j]).wait()

        def gather_issue(j, b):
            pltpu.async_copy(xl_hbm.at[src_vs[j]], xl_rows[b], sem_gl[b])
            pltpu.async_copy(xr_hbm.at[dst_vs[j]], xr_rows[b], sem_gr[b])

        def gather_wait(j, b):
            pltpu.make_async_copy(xl_hbm.at[src_vs[j]], xl_rows[b],
                                  sem_gl[b]).wait()
            pltpu.make_async_copy(xr_hbm.at[dst_vs[j]], xr_rows[b],
                                  sem_gr[b]).wait()

        def scatter_issue(j, b):
            pltpu.async_copy(msg_vs[b], acc_sh.at[dst_vs[j]], sem_sa[b],
                             add=True)
            pltpu.async_copy(ex_vs[b], den_sh.at[dst_vs[j]], sem_sd[b],
                             add=True)

        def scatter_wait(j, b):
            pltpu.make_async_copy(msg_vs[b], acc_sh.at[dst_vs[j]],
                                  sem_sa[b]).wait()
            pltpu.make_async_copy(ex_vs[b], den_sh.at[dst_vs[j]],
                                  sem_sd[b]).wait()

        def compute(b):
            xl_r, xr_r, msg_r, ex_r = (xl_rows[b], xr_rows[b],
                                       msg_vs[b], ex_vs[b])

            def g_body(g, gcarry):
                eidx = g * 16 + lane
                accs = [zero16] * 4
                for k in range(HID):
                    col = jnp.bitwise_and(lane + k, HID - 1)
                    att_w = att_v[pl.ds(k, 16)]
                    a = plsc.load_gather(xl_r, [eidx, col])
                    bb = plsc.load_gather(xr_r, [eidx, col])
                    sv = a + bb
                    sv = jnp.maximum(sv, 0.2 * sv)
                    accs[k % 4] = accs[k % 4] + att_w * sv
                ex = jnp.exp((accs[0] + accs[1]) + (accs[2] + accs[3]))
                ex_r[pl.ds(g * 16, 16)] = ex
                for k in range(HID):
                    col = jnp.bitwise_and(lane + k, HID - 1)
                    v = plsc.load_gather(xl_r, [eidx, col]) * ex
                    plsc.store_scatter(msg_r, [eidx, col], v)
                return gcarry
            lax.fori_loop(0, C // 16, g_body, 0)

        def slot(i, p):
            b = p % 2
            bn = (p + 1) % 2
            jn = (p + 1) % 4
            jnn = (p + 2) % 4

            @pl.when(i >= 2)
            def _():
                scatter_wait(jnn, b)

            @pl.when(i + 2 < n)
            def _():
                idx_issue(i + 2, jnn)

            @pl.when(i + 1 < n)
            def _():
                idx_wait(i + 1, jn)
                gather_issue(jn, bn)

            gather_wait(p, b)
            compute(b)
            scatter_issue(p, b)

        idx_issue(0, 0)
        idx_issue(1, 1)
        idx_wait(0, 0)
        gather_issue(0, 0)

        def quad(q, carry):
            i0 = q * 4
            for p in range(4):
                slot(i0 + p, p)
            return carry
        lax.fori_loop(0, n // 4, quad, 0)

        scatter_wait(2, 0)
        scatter_wait(3, 1)
        plsc.subcore_barrier()

        pltpu.sync_copy(acc_sh.at[pl.ds(s * RPT, RPT), :],
                        acc_out.at[c, pl.ds(s * RPT, RPT), :])
        pltpu.sync_copy(den_sh.at[pl.ds(s * RPT, RPT)],
                        den_out.at[c, pl.ds(s * RPT, RPT)])

    return pl.kernel(
        body,
        out_type=[jax.ShapeDtypeStruct((NC, N_ACC, HID), jnp.float32),
                  jax.ShapeDtypeStruct((NC, N_ACC), jnp.float32)],
        mesh=mesh,
        compiler_params=pltpu.CompilerParams(needs_layout_passes=False,
                                             use_tc_tiling_on_sc=False),
        scratch_types=(
            [pltpu.VMEM((C,), jnp.int32)] * 8 +
            [pltpu.VMEM((C, HID), jnp.float32)] * 6 +
            [pltpu.VMEM((C,), jnp.float32)] * 2 +
            [pltpu.VMEM((HID + 16,), jnp.float32),
             pltpu.VMEM((RPT,), jnp.float32),
             pltpu.VMEM_SHARED((N_ACC, HID), jnp.float32),
             pltpu.VMEM_SHARED((N_ACC,), jnp.float32)] +
            [pltpu.SemaphoreType.DMA] * 12
        ),
    )


def _gat_edges_sc(xl, xr, src, dst, att):
    e = src.shape[0]
    n_chunks = _ceil_div(_ceil_div(e, NW * C), 4) * 4
    e_pad = n_chunks * NW * C
    pad = e_pad - e
    if pad:
        src = jnp.concatenate([src, jnp.zeros((pad,), jnp.int32)])
        dst = jnp.concatenate([dst, jnp.full((pad,), PAD_ROW, jnp.int32)])
    return _make_gat_sc(e_pad, n_chunks)(xl, xr, src, dst, att)


_BLK = 1000


def _dot(a, b):
    return jnp.dot(a, b, preferred_element_type=jnp.float32)


def _full(shape):
    nd = len(shape)
    return pl.BlockSpec(shape, lambda i: (0,) * nd)


def _proj_tc(x, wl, bl, wr, br):
    n, k = x.shape
    h = wl.shape[1]

    def body(x_ref, wl_ref, bl_ref, wr_ref, br_ref, xl_ref, xr_ref):
        xb = x_ref[...]
        xl_ref[...] = _dot(xb, wl_ref[...]) + bl_ref[...]
        xr_ref[...] = _dot(xb, wr_ref[...]) + br_ref[...]

    return pl.pallas_call(
        body,
        grid=(n // _BLK,),
        in_specs=[pl.BlockSpec((_BLK, k), lambda i: (i, 0)),
                  _full((k, h)), _full((1, h)), _full((k, h)), _full((1, h))],
        out_specs=[pl.BlockSpec((_BLK, h), lambda i: (i, 0))] * 2,
        out_shape=[jax.ShapeDtypeStruct((n, h), jnp.float32)] * 2,
    )(x, wl, bl, wr, br)


def _combine(acc_ref, den_ref, bias_ref):
    a = acc_ref[0] + acc_ref[1]
    d = den_ref[0] + den_ref[1]
    return a / (d + 1e-16) + bias_ref[...]


def _combine_proj_tc(acc, den3, bias, wl, bl, wr, br, n):
    h_dim = HID

    def body(acc_ref, den_ref, bias_ref, wl_ref, bl_ref, wr_ref, br_ref,
             xl_ref, xr_ref):
        hb = _combine(acc_ref, den_ref, bias_ref)
        xl_ref[...] = _dot(hb, wl_ref[...]) + bl_ref[...]
        xr_ref[...] = _dot(hb, wr_ref[...]) + br_ref[...]

    return pl.pallas_call(
        body,
        grid=(n // _BLK,),
        in_specs=[pl.BlockSpec((NC, _BLK, HID), lambda i: (0, i, 0)),
                  pl.BlockSpec((NC, _BLK, 1), lambda i: (0, i, 0)),
                  _full((1, HID)),
                  _full((HID, h_dim)), _full((1, h_dim)),
                  _full((HID, h_dim)), _full((1, h_dim))],
        out_specs=[pl.BlockSpec((_BLK, h_dim), lambda i: (i, 0))] * 2,
        out_shape=[jax.ShapeDtypeStruct((n, h_dim), jnp.float32)] * 2,
    )(acc, den3, bias, wl, bl, wr, br)


def _combine_proj2_tc(acc, den3, bias, wl, bl, xf, wr, br, n):
    kf = xf.shape[1]

    def body(acc_ref, den_ref, bias_ref, wl_ref, bl_ref, xf_ref, wr_ref,
             br_ref, xl_ref, xr_ref):
        hb = _combine(acc_ref, den_ref, bias_ref)
        xl_ref[...] = _dot(hb, wl_ref[...]) + bl_ref[...]
        xr_ref[...] = _dot(xf_ref[...], wr_ref[...]) + br_ref[...]

    return pl.pallas_call(
        body,
        grid=(n // _BLK,),
        in_specs=[pl.BlockSpec((NC, _BLK, HID), lambda i: (0, i, 0)),
                  pl.BlockSpec((NC, _BLK, 1), lambda i: (0, i, 0)),
                  _full((1, HID)),
                  _full((HID, HID)), _full((1, HID)),
                  pl.BlockSpec((_BLK, kf), lambda i: (i, 0)),
                  _full((kf, HID)), _full((1, HID))],
        out_specs=[pl.BlockSpec((_BLK, HID), lambda i: (i, 0))] * 2,
        out_shape=[jax.ShapeDtypeStruct((n, HID), jnp.float32)] * 2,
    )(acc, den3, bias, wl, bl, xf, wr, br)


def _final_tc(acc, den3, bias, w1, b1, w2, b2, w3, b3,
              xf, wlin, blin, ws, bs, wc, bc, we, be, n):
    def body(acc_ref, den_ref, bias_ref, w1_ref, b1_ref, w2_ref, b2_ref,
             w3_ref, b3_ref, xf_ref, wlin_ref, blin_ref, ws_ref, bs_ref,
             wc_ref, bc_ref, we_ref, be_ref, out_ref):
        foot = _combine(acc_ref, den_ref, bias_ref)
        m = jnp.maximum(_dot(foot, w1_ref[...]) + b1_ref[...], 0.0)
        m = jnp.maximum(_dot(m, w2_ref[...]) + b2_ref[...], 0.0)
        m3 = _dot(m, w3_ref[...]) + b3_ref[...]
        xb = xf_ref[...]
        lin = _dot(xb, wlin_ref[...]) + blin_ref[...]
        a = jnp.maximum(_dot(xb, ws_ref[...]) + bs_ref[...], 0.0)
        a = jnp.maximum(_dot(a, wc_ref[...]) + bc_ref[...], 0.0)
        a = jnp.maximum(_dot(a, wc_ref[...]) + bc_ref[...], 0.0)
        a = _dot(a, we_ref[...]) + be_ref[...]
        out_ref[...] = lin + a + m3

    kf = xf.shape[1]
    return pl.pallas_call(
        body,
        grid=(n // _BLK,),
        in_specs=[pl.BlockSpec((NC, _BLK, HID), lambda i: (0, i, 0)),
                  pl.BlockSpec((NC, _BLK, 1), lambda i: (0, i, 0)),
                  _full((1, HID)),
                  _full((HID, HID)), _full((1, HID)),
                  _full((HID, HID)), _full((1, HID)),
                  _full((HID, 1)), _full((1, 1)),
                  pl.BlockSpec((_BLK, kf), lambda i: (i, 0)),
                  _full((kf, 1)), _full((1, 1)),
                  _full((kf, HID)), _full((1, HID)),
                  _full((HID, HID)), _full((1, HID)),
                  _full((HID, 1)), _full((1, 1))],
        out_specs=pl.BlockSpec((_BLK, 1), lambda i: (i, 0)),
        out_shape=jax.ShapeDtypeStruct((n, 1), jnp.float32),
    )(acc, den3, bias, w1, b1, w2, b2, w3, b3,
      xf, wlin, blin, ws, bs, wc, bc, we, be)


def kernel(x_pano, x_footprint, edge_index_pano, edge_index_rev, params):
    p = params
    n_pano = x_pano.shape[0]
    n_foot = x_footprint.shape[0]
    src0, dst0 = edge_index_pano[0], edge_index_pano[1]
    srcr, dstr = edge_index_rev[0], edge_index_rev[1]

    def row(b):
        return b.reshape(1, -1)

    xl0, xr0 = _proj_tc(x_pano, p['conv0_Wl'], row(p['conv0_bl']),
                        p['conv0_Wr'], row(p['conv0_br']))
    acc0, den0 = _gat_edges_sc(xl0, xr0, src0, dst0,
                               p['conv0_att'].reshape(HID))
    xl1, xr1 = _combine_proj_tc(acc0, den0[..., None], row(p['conv0_bias']),
                                p['conv1_Wl'], row(p['conv1_bl']),
                                p['conv1_Wr'], row(p['conv1_br']), n_pano)
    acc1, den1 = _gat_edges_sc(xl1, xr1, src0, dst0,
                               p['conv1_att'].reshape(HID))
    xlt, xrt = _combine_proj2_tc(acc1, den1[..., None], row(p['conv1_bias']),
                                 p['convt_Wl'], row(p['convt_bl']),
                                 x_footprint, p['convt_Wr'],
                                 row(p['convt_br']), n_pano)
    acct, dent = _gat_edges_sc(xlt, xrt, srcr, dstr,
                               p['convt_att'].reshape(HID))
    return _final_tc(acct, dent[..., None], row(p['convt_bias']),
                     p['mlp_W1'], row(p['mlp_b1']),
                     p['mlp_W2'], row(p['mlp_b2']),
                     p['mlp_W3'], row(p['mlp_b3']),
                     x_footprint,
                     p['null_W_lin'], row(p['null_b_lin']),
                     p['null_W_s'], row(p['null_b_s']),
                     p['null_W_c'], row(p['null_b_c']),
                     p['null_W_e'], row(p['null_b_e']), n_foot)
